# Initial kernel scaffold; baseline (speedup 1.0000x reference)
import jax
import jax.numpy as jnp
from jax.experimental import pallas as pl


def kernel(x, edge_index, W1, b1, W2, b2, W3, b3):
    raise NotImplementedError("write your pallas kernel here")


# speedup vs baseline: 7.7189x; 7.7189x over previous
import functools

import jax
import jax.numpy as jnp
from jax import lax
from jax.experimental import pallas as pl
from jax.experimental.pallas import tpu as pltpu
from jax.experimental.pallas import tpu_sc as plsc

N = 10000
NPAD = 10240
E = 320000
NC, NS = 2, 16
NW = NC * NS
K = 128
CHUNKS = 80
EPAD = NW * CHUNKS * K
ROWS_PER_TILE = NPAD // NS

@functools.cache
def _mesh():
    return plsc.VectorSubcoreMesh(
        core_axis_name="c", subcore_axis_name="s",
        num_cores=NC, num_subcores=NS)


def _deg_body(dst3d, ones_hbm, zeros_hbm, deg_out, dst_v, ones_v, shared):
    cid = lax.axis_index("c")
    sid = lax.axis_index("s")
    wid = cid * NS + sid
    r0 = sid * ROWS_PER_TILE
    pltpu.sync_copy(zeros_hbm.at[pl.ds(r0, ROWS_PER_TILE)],
                    shared.at[pl.ds(r0, ROWS_PER_TILE)])
    pltpu.sync_copy(dst3d.at[wid], dst_v)
    pltpu.sync_copy(ones_hbm, ones_v)
    plsc.subcore_barrier()

    def body(g, carry):
        pltpu.sync_copy(ones_v, shared.at[dst_v.at[g]], add=True)
        return carry

    lax.fori_loop(0, CHUNKS, body, 0)
    plsc.subcore_barrier()
    pltpu.sync_copy(shared.at[pl.ds(r0, ROWS_PER_TILE)],
                    deg_out.at[cid, pl.ds(r0, ROWS_PER_TILE)])


@functools.cache
def _deg_kernel():
    return pl.kernel(
        _deg_body,
        out_type=jax.ShapeDtypeStruct((NC, NPAD, 16), jnp.float32),
        mesh=_mesh(),
        compiler_params=pltpu.CompilerParams(use_tc_tiling_on_sc=False),
        scratch_types=[
            pltpu.VMEM((CHUNKS, K), jnp.int32),
            pltpu.VMEM((K, 16), jnp.float32),
            pltpu.VMEM_SHARED((NPAD, 16), jnp.float32),
        ],
    )


def _agg_body(y_hbm, src3d, dst3d, zeros_hbm, z_out, src_v, dst_v, rows_v,
              shared):
    cid = lax.axis_index("c")
    sid = lax.axis_index("s")
    wid = cid * NS + sid
    r0 = sid * ROWS_PER_TILE
    pltpu.sync_copy(zeros_hbm.at[pl.ds(r0, ROWS_PER_TILE)],
                    shared.at[pl.ds(r0, ROWS_PER_TILE)])
    pltpu.sync_copy(src3d.at[wid], src_v)
    pltpu.sync_copy(dst3d.at[wid], dst_v)
    plsc.subcore_barrier()

    def body(g, carry):
        pltpu.sync_copy(y_hbm.at[src_v.at[g]], rows_v)
        pltpu.sync_copy(rows_v, shared.at[dst_v.at[g]], add=True)
        return carry

    lax.fori_loop(0, CHUNKS, body, 0)
    plsc.subcore_barrier()
    pltpu.sync_copy(shared.at[pl.ds(r0, ROWS_PER_TILE)],
                    z_out.at[cid, pl.ds(r0, ROWS_PER_TILE)])


@functools.cache
def _make_agg(c):
    return pl.kernel(
        _agg_body,
        out_type=jax.ShapeDtypeStruct((NC, NPAD, c), jnp.float32),
        mesh=_mesh(),
        compiler_params=pltpu.CompilerParams(use_tc_tiling_on_sc=False),
        scratch_types=[
            pltpu.VMEM((CHUNKS, K), jnp.int32),
            pltpu.VMEM((CHUNKS, K), jnp.int32),
            pltpu.VMEM((K, c), jnp.float32),
            pltpu.VMEM_SHARED((NPAD, c), jnp.float32),
        ],
    )


BM = 512
GRID = NPAD // BM


def _dinv_of(deg_ref):
    return lax.rsqrt(deg_ref[0][:, :1] + deg_ref[1][:, :1] + 1.0)


def _stage_a_body(x_ref, w_ref, deg_ref, y_ref):
    dinv = _dinv_of(deg_ref)
    y_ref[...] = dinv * jnp.dot(x_ref[...], w_ref[...],
                                preferred_element_type=jnp.float32)


def _stage_b_body(z_ref, y_ref, deg_ref, b_ref, w_ref, o_ref):
    dinv = _dinv_of(deg_ref)
    h = jnp.maximum(dinv * (z_ref[0] + z_ref[1] + y_ref[...]) + b_ref[...],
                    0.0)
    o_ref[...] = dinv * jnp.dot(h, w_ref[...],
                                preferred_element_type=jnp.float32)


def _stage_c_body(z_ref, y_ref, deg_ref, b_ref, o_ref):
    dinv = _dinv_of(deg_ref)
    o_ref[...] = dinv * (z_ref[0] + z_ref[1] + y_ref[...]) + b_ref[...]


def _row_spec(c):
    return pl.BlockSpec((BM, c), lambda i: (i, 0))


def _z_spec(c):
    return pl.BlockSpec((NC, BM, c), lambda i: (0, i, 0))


_DEG_SPEC = pl.BlockSpec((NC, BM, 16), lambda i: (0, i, 0))


def _b_spec(c):
    return pl.BlockSpec((1, c), lambda i: (0, 0))


def _w_spec(ci, co):
    return pl.BlockSpec((ci, co), lambda i: (0, 0))


def _stage_a(xp, w, deg):
    return pl.pallas_call(
        _stage_a_body,
        grid=(GRID,),
        in_specs=[_row_spec(128), _w_spec(128, 128), _DEG_SPEC],
        out_specs=_row_spec(128),
        out_shape=jax.ShapeDtypeStruct((NPAD, 128), jnp.float32),
    )(xp, w, deg)


def _stage_b(z, y, deg, b, w, co):
    return pl.pallas_call(
        _stage_b_body,
        grid=(GRID,),
        in_specs=[_z_spec(128), _row_spec(128), _DEG_SPEC, _b_spec(128),
                  _w_spec(128, co)],
        out_specs=_row_spec(co),
        out_shape=jax.ShapeDtypeStruct((NPAD, co), jnp.float32),
    )(z, y, deg, b, w)


def _stage_c(z, y, deg, b):
    return pl.pallas_call(
        _stage_c_body,
        grid=(GRID,),
        in_specs=[_z_spec(64), _row_spec(64), _DEG_SPEC, _b_spec(64)],
        out_specs=_row_spec(64),
        out_shape=jax.ShapeDtypeStruct((NPAD, 64), jnp.float32),
    )(z, y, deg, b)


def kernel(x, edge_index, W1, b1, W2, b2, W3, b3):
    src = edge_index[0].astype(jnp.int32)
    dst = edge_index[1].astype(jnp.int32)
    pad = EPAD - E
    src3d = jnp.concatenate([src, jnp.zeros((pad,), jnp.int32)]
                            ).reshape(NW, CHUNKS, K)
    dst3d = jnp.concatenate([dst, jnp.full((pad,), N, jnp.int32)]
                            ).reshape(NW, CHUNKS, K)
    xp = jnp.pad(x, ((0, NPAD - N), (0, 0)))
    zeros16 = jnp.zeros((NPAD, 16), jnp.float32)
    zeros128 = jnp.zeros((NPAD, 128), jnp.float32)
    zeros64 = jnp.zeros((NPAD, 64), jnp.float32)
    ones16 = jnp.ones((K, 16), jnp.float32)
    b1r, b2r, b3r = (b.reshape(1, -1) for b in (b1, b2, b3))

    deg = _deg_kernel()(dst3d, ones16, zeros16)
    y1 = _stage_a(xp, W1, deg)
    z1 = _make_agg(128)(y1, src3d, dst3d, zeros128)
    y2 = _stage_b(z1, y1, deg, b1r, W2, 128)
    z2 = _make_agg(128)(y2, src3d, dst3d, zeros128)
    y3 = _stage_b(z2, y2, deg, b2r, W3, 64)
    z3 = _make_agg(64)(y3, src3d, dst3d, zeros64)
    out = _stage_c(z3, y3, deg, b3r)
    return out[:N]


# speedup vs baseline: 9.1992x; 1.1918x over previous
import functools

import jax
import jax.numpy as jnp
from jax import lax
from jax.experimental import pallas as pl
from jax.experimental.pallas import tpu as pltpu
from jax.experimental.pallas import tpu_sc as plsc

N = 10000
NPAD = 10240
E = 320000
NC, NS = 2, 16
NW = NC * NS
K = 128
CHUNKS = 80
PHASES = 2
CPP = CHUNKS // PHASES
EPAD = NW * CHUNKS * K
ROWS_PER_TILE = NPAD // NS

@functools.cache
def _mesh():
    return plsc.VectorSubcoreMesh(
        core_axis_name="c", subcore_axis_name="s",
        num_cores=NC, num_subcores=NS)


EPW = EPAD // NW


def _deg_body(dst2d, deg_out, dst_v, acc):
    cid = lax.axis_index("c")
    sid = lax.axis_index("s")
    wid = cid * NS + sid
    pltpu.sync_copy(dst2d.at[wid], dst_v)
    zeros = jnp.zeros((16,), jnp.float32)
    ones = jnp.ones((16,), jnp.float32)

    def zero(j, carry):
        acc[pl.ds(j * 16, 16)] = zeros
        return carry

    lax.fori_loop(0, NPAD // 16, zero, 0)

    def body(j, carry):
        idx = dst_v[pl.ds(j * 16, 16)]
        plsc.addupdate_scatter(acc, [idx], ones)
        return carry

    lax.fori_loop(0, EPW // 16, body, 0)
    pltpu.sync_copy(acc, deg_out.at[wid])


@functools.cache
def _deg_kernel():
    return pl.kernel(
        _deg_body,
        out_type=jax.ShapeDtypeStruct((NW, NPAD), jnp.float32),
        mesh=_mesh(),
        compiler_params=pltpu.CompilerParams(use_tc_tiling_on_sc=False,
                                             needs_layout_passes=False),
        scratch_types=[
            pltpu.VMEM((EPW,), jnp.int32),
            pltpu.VMEM((NPAD,), jnp.float32),
        ],
    )


def _agg_body(y_hbm, src3d, dst3d, zeros_hbm, z_out, src_v, dst_v, rows_v,
              shared, sem0, sem1):
    cid = lax.axis_index("c")
    sid = lax.axis_index("s")
    wid = cid * NS + sid
    r0 = sid * ROWS_PER_TILE
    pltpu.sync_copy(zeros_hbm.at[pl.ds(r0, ROWS_PER_TILE)],
                    shared.at[pl.ds(r0, ROWS_PER_TILE)])
    plsc.subcore_barrier()

    def phase(p, carry):
        pltpu.sync_copy(src3d.at[wid, pl.ds(p * CPP, CPP)], src_v)
        pltpu.sync_copy(dst3d.at[wid, pl.ds(p * CPP, CPP)], dst_v)
        pltpu.async_copy(y_hbm.at[src_v.at[0]], rows_v.at[0], sem0)

        def body(i, c2):
            g0 = 2 * i
            pltpu.async_copy(y_hbm.at[src_v.at[g0 + 1]], rows_v.at[1], sem1)
            pltpu.make_async_copy(y_hbm.at[src_v.at[g0]], rows_v.at[0],
                                  sem0).wait()
            pltpu.sync_copy(rows_v.at[0], shared.at[dst_v.at[g0]], add=True)

            @pl.when(g0 + 2 < CPP)
            def _():
                pltpu.async_copy(y_hbm.at[src_v.at[g0 + 2]], rows_v.at[0],
                                 sem0)

            pltpu.make_async_copy(y_hbm.at[src_v.at[g0 + 1]], rows_v.at[1],
                                  sem1).wait()
            pltpu.sync_copy(rows_v.at[1], shared.at[dst_v.at[g0 + 1]],
                            add=True)
            return c2

        lax.fori_loop(0, CPP // 2, body, 0)
        return carry

    lax.fori_loop(0, PHASES, phase, 0)
    plsc.subcore_barrier()
    pltpu.sync_copy(shared.at[pl.ds(r0, ROWS_PER_TILE)],
                    z_out.at[cid, pl.ds(r0, ROWS_PER_TILE)])


@functools.cache
def _make_agg(c):
    return pl.kernel(
        _agg_body,
        out_type=jax.ShapeDtypeStruct((NC, NPAD, c), jnp.float32),
        mesh=_mesh(),
        compiler_params=pltpu.CompilerParams(use_tc_tiling_on_sc=False),
        scratch_types=[
            pltpu.VMEM((CPP, K), jnp.int32),
            pltpu.VMEM((CPP, K), jnp.int32),
            pltpu.VMEM((2, K, c), jnp.float32),
            pltpu.VMEM_SHARED((NPAD, c), jnp.float32),
            pltpu.SemaphoreType.DMA,
            pltpu.SemaphoreType.DMA,
        ],
    )


BM = 512
GRID = NPAD // BM


def _dinv_of(deg_ref):
    deg = jnp.sum(deg_ref[...], axis=0)
    return lax.rsqrt(deg + 1.0)[:, None]


def _stage_a_body(x_ref, w_ref, deg_ref, y_ref):
    dinv = _dinv_of(deg_ref)
    y_ref[...] = dinv * jnp.dot(x_ref[...], w_ref[...],
                                preferred_element_type=jnp.float32)


def _stage_b_body(z_ref, y_ref, deg_ref, b_ref, w_ref, o_ref):
    dinv = _dinv_of(deg_ref)
    h = jnp.maximum(dinv * (z_ref[0] + z_ref[1] + y_ref[...]) + b_ref[...],
                    0.0)
    o_ref[...] = dinv * jnp.dot(h, w_ref[...],
                                preferred_element_type=jnp.float32)


def _stage_c_body(z_ref, y_ref, deg_ref, b_ref, o_ref):
    dinv = _dinv_of(deg_ref)
    o_ref[...] = dinv * (z_ref[0] + z_ref[1] + y_ref[...]) + b_ref[...]


def _row_spec(c):
    return pl.BlockSpec((BM, c), lambda i: (i, 0))


def _z_spec(c):
    return pl.BlockSpec((NC, BM, c), lambda i: (0, i, 0))


_DEG_SPEC = pl.BlockSpec((NW, BM), lambda i: (0, i))


def _b_spec(c):
    return pl.BlockSpec((1, c), lambda i: (0, 0))


def _w_spec(ci, co):
    return pl.BlockSpec((ci, co), lambda i: (0, 0))


def _stage_a(xp, w, deg):
    return pl.pallas_call(
        _stage_a_body,
        grid=(GRID,),
        in_specs=[_row_spec(128), _w_spec(128, 128), _DEG_SPEC],
        out_specs=_row_spec(128),
        out_shape=jax.ShapeDtypeStruct((NPAD, 128), jnp.float32),
    )(xp, w, deg)


def _stage_b(z, y, deg, b, w, co):
    return pl.pallas_call(
        _stage_b_body,
        grid=(GRID,),
        in_specs=[_z_spec(128), _row_spec(128), _DEG_SPEC, _b_spec(128),
                  _w_spec(128, co)],
        out_specs=_row_spec(co),
        out_shape=jax.ShapeDtypeStruct((NPAD, co), jnp.float32),
    )(z, y, deg, b, w)


def _stage_c(z, y, deg, b):
    return pl.pallas_call(
        _stage_c_body,
        grid=(GRID,),
        in_specs=[_z_spec(64), _row_spec(64), _DEG_SPEC, _b_spec(64)],
        out_specs=_row_spec(64),
        out_shape=jax.ShapeDtypeStruct((NPAD, 64), jnp.float32),
    )(z, y, deg, b)


def kernel(x, edge_index, W1, b1, W2, b2, W3, b3):
    src = edge_index[0].astype(jnp.int32)
    dst = edge_index[1].astype(jnp.int32)
    pad = EPAD - E
    src3d = jnp.concatenate([src, jnp.zeros((pad,), jnp.int32)]
                            ).reshape(NW, CHUNKS, K)
    dst3d = jnp.concatenate([dst, jnp.full((pad,), N, jnp.int32)]
                            ).reshape(NW, CHUNKS, K)
    xp = jnp.pad(x, ((0, NPAD - N), (0, 0)))
    zeros128 = jnp.zeros((NPAD, 128), jnp.float32)
    zeros64 = jnp.zeros((NPAD, 64), jnp.float32)
    b1r, b2r, b3r = (b.reshape(1, -1) for b in (b1, b2, b3))

    deg = _deg_kernel()(dst3d.reshape(NW, EPW))
    y1 = _stage_a(xp, W1, deg)
    z1 = _make_agg(128)(y1, src3d, dst3d, zeros128)
    y2 = _stage_b(z1, y1, deg, b1r, W2, 128)
    z2 = _make_agg(128)(y2, src3d, dst3d, zeros128)
    y3 = _stage_b(z2, y2, deg, b2r, W3, 64)
    z3 = _make_agg(64)(y3, src3d, dst3d, zeros64)
    out = _stage_c(z3, y3, deg, b3r)
    return out[:N]
